# baseline (speedup 1.0000x reference)
import jax
import jax.numpy as jnp
from jax.experimental import pallas as pl

_B = 16
_N = 512
_P = 256
_S = 32
_PT = 32
_NT = _P // _PT
_R = _S * _PT
_BIGF = 1e30
_BIGI = 1 << 30


def _knn_kernel(pts_ref, ctr_ref, ptsm_ref, nbr_ref):
    prow = [pts_ref[0, c:c + 1, :] for c in range(3)]
    ccol = [ctr_ref[0, c] for c in range(3)]
    d2 = ((ccol[0] - prow[0]) ** 2 + (ccol[1] - prow[1]) ** 2
          + (ccol[2] - prow[2]) ** 2)
    lane = jax.lax.broadcasted_iota(jnp.int32, (_P, _N), 1)
    ptsm = ptsm_ref[0]
    for k in range(_S):
        m = jnp.min(d2, axis=1, keepdims=True)
        idxm = jnp.where(d2 == m, lane, _BIGI)
        win = jnp.min(idxm, axis=1, keepdims=True)
        onehot = lane == win
        sel = jnp.dot(onehot.astype(jnp.float32), ptsm,
                      preferred_element_type=jnp.float32,
                      precision=jax.lax.Precision.HIGHEST)
        for c in range(3):
            nbr_ref[0, c, k * _P:(k + 1) * _P, :] = sel[:, c:c + 1]
        d2 = jnp.where(onehot, _BIGF, d2)


def _lin3(nbr_ref, ctr_ref, w3t_ref):
    nbr = jnp.reshape(nbr_ref[...], (3, _R, 1))
    ctr = jnp.reshape(ctr_ref[...], (3, _R, 1))
    big = jnp.concatenate(
        [jnp.concatenate([nbr[c], nbr[c] - ctr[c]], axis=1)
         for c in range(3)], axis=0)
    v = jnp.dot(big.astype(jnp.bfloat16), w3t_ref[...],
                preferred_element_type=jnp.float32)
    return [v[c * _R:(c + 1) * _R, :] for c in range(3)]


def _vn_block(v, dmat_t_ref, mv, gb):
    n = jnp.sqrt((v[0] * v[0] + v[1] * v[1] + v[2] * v[2]) + 1e-12)
    nbn = (gb[0:1, :] * (n - mv[0:1, :])) / jnp.sqrt(mv[1:2, :] + 1e-5) \
        + gb[1:2, :]
    vb = [vc / n * nbn for vc in v]
    big = jnp.concatenate(vb, axis=0)
    d = jnp.dot(big.astype(jnp.bfloat16), dmat_t_ref[...],
                preferred_element_type=jnp.float32)
    dc = [d[c * _R:(c + 1) * _R, :] for c in range(3)]
    dot = (vb[0] * dc[0] + vb[1] * dc[1]) + vb[2] * dc[2]
    dsq = (dc[0] * dc[0] + dc[1] * dc[1]) + dc[2] * dc[2]
    w = dot / (dsq + 1e-8)
    pos = dot >= 0
    return [0.2 * vb[c] + 0.8 * jnp.where(pos, vb[c], vb[c] - w * dc[c])
            for c in range(3)]


def _norm3(v):
    return jnp.sqrt(((v[0] * v[0] + v[1] * v[1]) + v[2] * v[2]) + 1e-12)


def _to_u4(nbr_ref, ctr_ref, w3t_ref, d3t_ref, mv3_ref, bn3_ref, w4t_ref):
    v3 = _lin3(nbr_ref, ctr_ref, w3t_ref)
    o3 = _vn_block(v3, d3t_ref, mv3_ref[...], bn3_ref[...])
    big = jnp.concatenate(o3, axis=0)
    u = jnp.dot(big.astype(jnp.bfloat16), w4t_ref[...],
                preferred_element_type=jnp.float32)
    return [u[c * _R:(c + 1) * _R, :] for c in range(3)]


def _n3_kernel(nbr_ref, ctr_ref, w3t_ref, out_ref):
    v3 = _lin3(nbr_ref, ctr_ref, w3t_ref)
    out_ref[0] = jnp.reshape(_norm3(v3), (_S, _PT, 128))


def _n4_kernel(nbr_ref, ctr_ref, w3t_ref, d3t_ref, mv3_ref, bn3_ref,
               w4t_ref, out_ref):
    u = _to_u4(nbr_ref, ctr_ref, w3t_ref, d3t_ref, mv3_ref, bn3_ref,
               w4t_ref)
    out_ref[0] = jnp.reshape(_norm3(u), (_S, _PT, 256))


def _final_kernel(nbr_ref, ctr_ref, w3t_ref, d3t_ref, mv3_ref, bn3_ref,
                  w4t_ref, d4t_ref, mv4_ref, bn4_ref, out_ref):
    u = _to_u4(nbr_ref, ctr_ref, w3t_ref, d3t_ref, mv3_ref, bn3_ref,
               w4t_ref)
    o4 = _vn_block(u, d4t_ref, mv4_ref[...], bn4_ref[...])
    oc = [jnp.reshape(o4[c], (_S, _PT, 256)) for c in range(3)]
    n2 = (oc[0] * oc[0] + oc[1] * oc[1]) + oc[2] * oc[2]
    best = n2[0]
    bv = [oc[c][0] for c in range(3)]
    for s in range(1, _S):
        upd = n2[s] > best
        best = jnp.where(upd, n2[s], best)
        bv = [jnp.where(upd, oc[c][s], bv[c]) for c in range(3)]
    for c in range(3):
        out_ref[0, c] = bv[c]


def _bfr(x):
    return x.astype(jnp.bfloat16).astype(jnp.float32)


def _pool_basis_kernel(pool_ref, wp_ref, out_ref):
    p = [pool_ref[0, c] for c in range(3)]
    n2 = (p[0] * p[0] + p[1] * p[1]) + p[2] * p[2]
    m = jnp.max(n2, axis=0, keepdims=True)
    sub = jax.lax.broadcasted_iota(jnp.int32, (_P, 256), 0)
    idxm = jnp.where(n2 == m, sub, _BIGI)
    win = jnp.min(idxm, axis=0, keepdims=True)
    onehot = sub == win
    out_ref[...] = jnp.zeros_like(out_ref)
    for c in range(3):
        feat = _bfr(jnp.sum(jnp.where(onehot, p[c], 0.0), axis=0,
                            keepdims=True))
        v1 = jnp.sum(feat * wp_ref[0:1, :], axis=1, keepdims=True)
        v2 = jnp.sum(feat * wp_ref[1:2, :], axis=1, keepdims=True)
        out_ref[0, c:c + 1, :] = jnp.broadcast_to(v1, (1, 128))
        out_ref[0, c + 3:c + 4, :] = jnp.broadcast_to(v2, (1, 128))


def _pad8(rows):
    return jnp.concatenate(
        rows + [jnp.zeros((8 - len(rows), rows[0].shape[1]), jnp.float32)],
        axis=0)


@jax.jit
def kernel(x, W1, D1, g1, b1, W2, D2, g2, b2, W3, D3, g3, b3, W4, D4, g4,
           b4, Wp):
    del W1, D1, g1, b1, W2, D2, g2, b2
    f32 = jnp.float32
    bf = jnp.bfloat16

    xt = jnp.transpose(x, (0, 2, 1))
    pts_t = xt[:, ::4, :]
    pts = jnp.transpose(pts_t, (0, 2, 1)).astype(f32)
    ctrcol = pts[:, :, ::2, None]
    ctr5 = jnp.broadcast_to(ctrcol[:, :, None, :, :], (_B, 3, _S, _P, 1))

    bn3 = _pad8([g3[None, :], b3[None, :]])
    bn4 = _pad8([g4[None, :], b4[None, :]])
    wpb = Wp.astype(bf).astype(f32)
    wp = _pad8([wpb[0][None, :], wpb[1][None, :]])
    w3t = jnp.transpose(W3).astype(bf)
    d3t = jnp.transpose(D3).astype(bf)
    w4t = jnp.transpose(W4).astype(bf)
    d4t = jnp.transpose(D4).astype(bf)

    ptsm = jnp.concatenate(
        [pts_t.astype(f32), jnp.zeros((_B, _N, 5), f32)], axis=-1)

    nbr = pl.pallas_call(
        _knn_kernel,
        grid=(_B,),
        in_specs=[
            pl.BlockSpec((1, 3, _N), lambda b: (b, 0, 0)),
            pl.BlockSpec((1, 3, _P, 1), lambda b: (b, 0, 0, 0)),
            pl.BlockSpec((1, _N, 8), lambda b: (b, 0, 0)),
        ],
        out_specs=pl.BlockSpec((1, 3, _P * _S, 1), lambda b: (b, 0, 0, 0)),
        out_shape=jax.ShapeDtypeStruct((_B, 3, _P * _S, 1), f32),
    )(pts, ctrcol, ptsm)

    nbr5 = jnp.reshape(nbr, (_B, 3, _S, _P, 1))
    nbr_spec = pl.BlockSpec((1, 3, _S, _PT, 1), lambda b, t: (b, 0, 0, t, 0))
    w3t_spec = pl.BlockSpec((2, 128), lambda b, t: (0, 0))
    d3t_spec = pl.BlockSpec((128, 128), lambda b, t: (0, 0))
    w4t_spec = pl.BlockSpec((128, 256), lambda b, t: (0, 0))
    d4t_spec = pl.BlockSpec((256, 256), lambda b, t: (0, 0))
    r128_spec = pl.BlockSpec((8, 128), lambda b, t: (0, 0))
    r256_spec = pl.BlockSpec((8, 256), lambda b, t: (0, 0))

    n3d = pl.pallas_call(
        _n3_kernel,
        grid=(_B, _NT),
        in_specs=[nbr_spec, nbr_spec, w3t_spec],
        out_specs=pl.BlockSpec((1, _S, _PT, 128), lambda b, t: (b, 0, t, 0)),
        out_shape=jax.ShapeDtypeStruct((_B, _S, _P, 128), f32),
    )(nbr5, ctr5, w3t)

    n3s = jnp.transpose(n3d, (0, 3, 2, 1))
    mv3 = _pad8([jnp.mean(n3s, axis=(0, 2, 3))[None, :],
                 jnp.var(n3s, axis=(0, 2, 3))[None, :]])

    n4d = pl.pallas_call(
        _n4_kernel,
        grid=(_B, _NT),
        in_specs=[nbr_spec, nbr_spec, w3t_spec, d3t_spec, r128_spec,
                  r128_spec, w4t_spec],
        out_specs=pl.BlockSpec((1, _S, _PT, 256), lambda b, t: (b, 0, t, 0)),
        out_shape=jax.ShapeDtypeStruct((_B, _S, _P, 256), f32),
    )(nbr5, ctr5, w3t, d3t, mv3, bn3, w4t)

    n4s = jnp.transpose(n4d, (0, 3, 2, 1))
    mv4 = _pad8([jnp.mean(n4s, axis=(0, 2, 3))[None, :],
                 jnp.var(n4s, axis=(0, 2, 3))[None, :]])

    pooled = pl.pallas_call(
        _final_kernel,
        grid=(_B, _NT),
        in_specs=[nbr_spec, nbr_spec, w3t_spec, d3t_spec, r128_spec,
                  r128_spec, w4t_spec, d4t_spec, r256_spec, r256_spec],
        out_specs=pl.BlockSpec((1, 3, _PT, 256), lambda b, t: (b, 0, t, 0)),
        out_shape=jax.ShapeDtypeStruct((_B, 3, _P, 256), f32),
    )(nbr5, ctr5, w3t, d3t, mv3, bn3, w4t, d4t, mv4, bn4)

    basis = pl.pallas_call(
        _pool_basis_kernel,
        grid=(_B,),
        in_specs=[
            pl.BlockSpec((1, 3, _P, 256), lambda b: (b, 0, 0, 0)),
            pl.BlockSpec((8, 256), lambda b: (0, 0)),
        ],
        out_specs=pl.BlockSpec((1, 8, 128), lambda b: (b, 0, 0)),
        out_shape=jax.ShapeDtypeStruct((_B, 8, 128), f32),
    )(pooled, wp)

    v1 = basis[:, 0:3, 0]
    v2 = basis[:, 3:6, 0]
    u1 = v1 / jnp.maximum(jnp.linalg.norm(v1, axis=-1, keepdims=True), 1e-8)
    u2_raw = v2 - jnp.sum(v2 * u1, axis=-1, keepdims=True) * u1
    u2 = u2_raw / jnp.maximum(jnp.linalg.norm(u2_raw, axis=-1, keepdims=True),
                              1e-6)
    u3 = jnp.cross(u1, u2)
    R = jnp.stack([u1, u2, u3], axis=-1)
    det = jnp.linalg.det(R)
    sign = jnp.sign(det)[:, None, None]
    R = jnp.concatenate([R[:, :, 0:1], R[:, :, 1:2], R[:, :, 2:3] * sign],
                        axis=-1)
    return (R, v1, v2)


# speedup vs baseline: 1.1408x; 1.1408x over previous
import jax
import jax.numpy as jnp
from jax.experimental import pallas as pl

_B = 16
_N = 512
_P = 256
_S = 32
_PT = 64
_NT = _P // _PT
_R = _S * _PT
_BIGF = 1e30
_BIGI = 1 << 30


def _knn_kernel(pts_ref, ctr_ref, nbr_ref):
    prow = [pts_ref[0, c:c + 1, :] for c in range(3)]
    ccol = [ctr_ref[0, c] for c in range(3)]
    d2 = ((ccol[0] - prow[0]) ** 2 + (ccol[1] - prow[1]) ** 2
          + (ccol[2] - prow[2]) ** 2)
    lane = jax.lax.broadcasted_iota(jnp.int32, (_P, _N), 1)
    for k in range(_S):
        m = jnp.min(d2, axis=1, keepdims=True)
        idxm = jnp.where(d2 == m, lane, _BIGI)
        win = jnp.min(idxm, axis=1, keepdims=True)
        onehot = lane == win
        for c in range(3):
            nbr_ref[0, c, k * _P:(k + 1) * _P, :] = jnp.sum(
                jnp.where(onehot, jnp.broadcast_to(prow[c], (_P, _N)), 0.0),
                axis=1, keepdims=True)
        d2 = jnp.where(onehot, _BIGF, d2)


def _lin3(nbr_ref, ctr_ref, w3t_ref):
    nbr = jnp.reshape(nbr_ref[...], (3, _R, 1))
    ctr = jnp.reshape(ctr_ref[...], (3, _R, 1))
    big = jnp.concatenate(
        [jnp.concatenate([nbr[c], nbr[c] - ctr[c]], axis=1)
         for c in range(3)], axis=0)
    v = jnp.dot(big.astype(jnp.bfloat16), w3t_ref[...],
                preferred_element_type=jnp.float32)
    return [v[c * _R:(c + 1) * _R, :] for c in range(3)]


def _vn_block(v, dmat_t_ref, mv, gb):
    n = jnp.sqrt((v[0] * v[0] + v[1] * v[1] + v[2] * v[2]) + 1e-12)
    nbn = (gb[0:1, :] * (n - mv[0:1, :])) / jnp.sqrt(mv[1:2, :] + 1e-5) \
        + gb[1:2, :]
    vb = [vc / n * nbn for vc in v]
    big = jnp.concatenate(vb, axis=0)
    d = jnp.dot(big.astype(jnp.bfloat16), dmat_t_ref[...],
                preferred_element_type=jnp.float32)
    dc = [d[c * _R:(c + 1) * _R, :] for c in range(3)]
    dot = (vb[0] * dc[0] + vb[1] * dc[1]) + vb[2] * dc[2]
    dsq = (dc[0] * dc[0] + dc[1] * dc[1]) + dc[2] * dc[2]
    w = dot / (dsq + 1e-8)
    pos = dot >= 0
    return [0.2 * vb[c] + 0.8 * jnp.where(pos, vb[c], vb[c] - w * dc[c])
            for c in range(3)]


def _norm3(v):
    return jnp.sqrt(((v[0] * v[0] + v[1] * v[1]) + v[2] * v[2]) + 1e-12)


def _to_u4(nbr_ref, ctr_ref, w3t_ref, d3t_ref, mv3_ref, bn3_ref, w4t_ref):
    v3 = _lin3(nbr_ref, ctr_ref, w3t_ref)
    o3 = _vn_block(v3, d3t_ref, mv3_ref[...], bn3_ref[...])
    big = jnp.concatenate(o3, axis=0)
    u = jnp.dot(big.astype(jnp.bfloat16), w4t_ref[...],
                preferred_element_type=jnp.float32)
    return [u[c * _R:(c + 1) * _R, :] for c in range(3)]


def _n3_kernel(nbr_ref, ctr_ref, w3t_ref, out_ref):
    v3 = _lin3(nbr_ref, ctr_ref, w3t_ref)
    out_ref[0] = jnp.reshape(_norm3(v3), (_S, _PT, 128))


def _n4_kernel(nbr_ref, ctr_ref, w3t_ref, d3t_ref, mv3_ref, bn3_ref,
               w4t_ref, out_ref):
    u = _to_u4(nbr_ref, ctr_ref, w3t_ref, d3t_ref, mv3_ref, bn3_ref,
               w4t_ref)
    out_ref[0] = jnp.reshape(_norm3(u), (_S, _PT, 256))


def _final_kernel(nbr_ref, ctr_ref, w3t_ref, d3t_ref, mv3_ref, bn3_ref,
                  w4t_ref, d4t_ref, mv4_ref, bn4_ref, out_ref):
    u = _to_u4(nbr_ref, ctr_ref, w3t_ref, d3t_ref, mv3_ref, bn3_ref,
               w4t_ref)
    o4 = _vn_block(u, d4t_ref, mv4_ref[...], bn4_ref[...])
    oc = [jnp.reshape(o4[c], (_S, _PT, 256)) for c in range(3)]
    n2 = (oc[0] * oc[0] + oc[1] * oc[1]) + oc[2] * oc[2]
    best = n2[0]
    bv = [oc[c][0] for c in range(3)]
    for s in range(1, _S):
        upd = n2[s] > best
        best = jnp.where(upd, n2[s], best)
        bv = [jnp.where(upd, oc[c][s], bv[c]) for c in range(3)]
    for c in range(3):
        out_ref[0, c] = bv[c]


def _bfr(x):
    return x.astype(jnp.bfloat16).astype(jnp.float32)


def _pool_basis_kernel(pool_ref, wp_ref, out_ref):
    p = [pool_ref[0, c] for c in range(3)]
    n2 = (p[0] * p[0] + p[1] * p[1]) + p[2] * p[2]
    m = jnp.max(n2, axis=0, keepdims=True)
    sub = jax.lax.broadcasted_iota(jnp.int32, (_P, 256), 0)
    idxm = jnp.where(n2 == m, sub, _BIGI)
    win = jnp.min(idxm, axis=0, keepdims=True)
    onehot = sub == win
    out_ref[...] = jnp.zeros_like(out_ref)
    for c in range(3):
        feat = _bfr(jnp.sum(jnp.where(onehot, p[c], 0.0), axis=0,
                            keepdims=True))
        v1 = jnp.sum(feat * wp_ref[0:1, :], axis=1, keepdims=True)
        v2 = jnp.sum(feat * wp_ref[1:2, :], axis=1, keepdims=True)
        out_ref[0, c:c + 1, :] = jnp.broadcast_to(v1, (1, 128))
        out_ref[0, c + 3:c + 4, :] = jnp.broadcast_to(v2, (1, 128))


def _pad8(rows):
    return jnp.concatenate(
        rows + [jnp.zeros((8 - len(rows), rows[0].shape[1]), jnp.float32)],
        axis=0)


@jax.jit
def kernel(x, W1, D1, g1, b1, W2, D2, g2, b2, W3, D3, g3, b3, W4, D4, g4,
           b4, Wp):
    del W1, D1, g1, b1, W2, D2, g2, b2
    f32 = jnp.float32
    bf = jnp.bfloat16

    xt = jnp.transpose(x, (0, 2, 1))
    pts_t = xt[:, ::4, :]
    pts = jnp.transpose(pts_t, (0, 2, 1)).astype(f32)
    ctrcol = pts[:, :, ::2, None]
    ctr5 = jnp.broadcast_to(ctrcol[:, :, None, :, :], (_B, 3, _S, _P, 1))

    bn3 = _pad8([g3[None, :], b3[None, :]])
    bn4 = _pad8([g4[None, :], b4[None, :]])
    wpb = Wp.astype(bf).astype(f32)
    wp = _pad8([wpb[0][None, :], wpb[1][None, :]])
    w3t = jnp.transpose(W3).astype(bf)
    d3t = jnp.transpose(D3).astype(bf)
    w4t = jnp.transpose(W4).astype(bf)
    d4t = jnp.transpose(D4).astype(bf)

    nbr = pl.pallas_call(
        _knn_kernel,
        grid=(_B,),
        in_specs=[
            pl.BlockSpec((1, 3, _N), lambda b: (b, 0, 0)),
            pl.BlockSpec((1, 3, _P, 1), lambda b: (b, 0, 0, 0)),
        ],
        out_specs=pl.BlockSpec((1, 3, _P * _S, 1), lambda b: (b, 0, 0, 0)),
        out_shape=jax.ShapeDtypeStruct((_B, 3, _P * _S, 1), f32),
    )(pts, ctrcol)

    nbr5 = jnp.reshape(nbr, (_B, 3, _S, _P, 1))
    nbr_spec = pl.BlockSpec((1, 3, _S, _PT, 1), lambda b, t: (b, 0, 0, t, 0))
    w3t_spec = pl.BlockSpec((2, 128), lambda b, t: (0, 0))
    d3t_spec = pl.BlockSpec((128, 128), lambda b, t: (0, 0))
    w4t_spec = pl.BlockSpec((128, 256), lambda b, t: (0, 0))
    d4t_spec = pl.BlockSpec((256, 256), lambda b, t: (0, 0))
    r128_spec = pl.BlockSpec((8, 128), lambda b, t: (0, 0))
    r256_spec = pl.BlockSpec((8, 256), lambda b, t: (0, 0))

    n3d = pl.pallas_call(
        _n3_kernel,
        grid=(_B, _NT),
        in_specs=[nbr_spec, nbr_spec, w3t_spec],
        out_specs=pl.BlockSpec((1, _S, _PT, 128), lambda b, t: (b, 0, t, 0)),
        out_shape=jax.ShapeDtypeStruct((_B, _S, _P, 128), f32),
    )(nbr5, ctr5, w3t)

    n3s = jnp.transpose(n3d, (0, 3, 2, 1))
    mv3 = _pad8([jnp.mean(n3s, axis=(0, 2, 3))[None, :],
                 jnp.var(n3s, axis=(0, 2, 3))[None, :]])

    n4d = pl.pallas_call(
        _n4_kernel,
        grid=(_B, _NT),
        in_specs=[nbr_spec, nbr_spec, w3t_spec, d3t_spec, r128_spec,
                  r128_spec, w4t_spec],
        out_specs=pl.BlockSpec((1, _S, _PT, 256), lambda b, t: (b, 0, t, 0)),
        out_shape=jax.ShapeDtypeStruct((_B, _S, _P, 256), f32),
    )(nbr5, ctr5, w3t, d3t, mv3, bn3, w4t)

    n4s = jnp.transpose(n4d, (0, 3, 2, 1))
    mv4 = _pad8([jnp.mean(n4s, axis=(0, 2, 3))[None, :],
                 jnp.var(n4s, axis=(0, 2, 3))[None, :]])

    pooled = pl.pallas_call(
        _final_kernel,
        grid=(_B, _NT),
        in_specs=[nbr_spec, nbr_spec, w3t_spec, d3t_spec, r128_spec,
                  r128_spec, w4t_spec, d4t_spec, r256_spec, r256_spec],
        out_specs=pl.BlockSpec((1, 3, _PT, 256), lambda b, t: (b, 0, t, 0)),
        out_shape=jax.ShapeDtypeStruct((_B, 3, _P, 256), f32),
    )(nbr5, ctr5, w3t, d3t, mv3, bn3, w4t, d4t, mv4, bn4)

    basis = pl.pallas_call(
        _pool_basis_kernel,
        grid=(_B,),
        in_specs=[
            pl.BlockSpec((1, 3, _P, 256), lambda b: (b, 0, 0, 0)),
            pl.BlockSpec((8, 256), lambda b: (0, 0)),
        ],
        out_specs=pl.BlockSpec((1, 8, 128), lambda b: (b, 0, 0)),
        out_shape=jax.ShapeDtypeStruct((_B, 8, 128), f32),
    )(pooled, wp)

    v1 = basis[:, 0:3, 0]
    v2 = basis[:, 3:6, 0]
    u1 = v1 / jnp.maximum(jnp.linalg.norm(v1, axis=-1, keepdims=True), 1e-8)
    u2_raw = v2 - jnp.sum(v2 * u1, axis=-1, keepdims=True) * u1
    u2 = u2_raw / jnp.maximum(jnp.linalg.norm(u2_raw, axis=-1, keepdims=True),
                              1e-6)
    u3 = jnp.cross(u1, u2)
    R = jnp.stack([u1, u2, u3], axis=-1)
    det = jnp.linalg.det(R)
    sign = jnp.sign(det)[:, None, None]
    R = jnp.concatenate([R[:, :, 0:1], R[:, :, 1:2], R[:, :, 2:3] * sign],
                        axis=-1)
    return (R, v1, v2)


# speedup vs baseline: 1.1566x; 1.0138x over previous
import jax
import jax.numpy as jnp
from jax.experimental import pallas as pl

_B = 16
_N = 512
_P = 256
_S = 32
_PT = 128
_NT = _P // _PT
_R = _S * _PT
_BIGF = 1e30
_BIGI = 1 << 30


def _knn_kernel(pts_ref, ctr_ref, nbr_ref):
    prow = [pts_ref[0, c:c + 1, :] for c in range(3)]
    ccol = [ctr_ref[0, c] for c in range(3)]
    d2 = ((ccol[0] - prow[0]) ** 2 + (ccol[1] - prow[1]) ** 2
          + (ccol[2] - prow[2]) ** 2)
    lane = jax.lax.broadcasted_iota(jnp.int32, (_P, _N), 1)
    for k in range(_S):
        m = jnp.min(d2, axis=1, keepdims=True)
        idxm = jnp.where(d2 == m, lane, _BIGI)
        win = jnp.min(idxm, axis=1, keepdims=True)
        onehot = lane == win
        for c in range(3):
            nbr_ref[0, c, k * _P:(k + 1) * _P, :] = jnp.sum(
                jnp.where(onehot, jnp.broadcast_to(prow[c], (_P, _N)), 0.0),
                axis=1, keepdims=True)
        d2 = jnp.where(onehot, _BIGF, d2)


def _lin3(nbr_ref, ctr_ref, w3t_ref):
    nbr = jnp.reshape(nbr_ref[...], (3, _R, 1))
    ctr = jnp.reshape(ctr_ref[...], (3, _R, 1))
    big = jnp.concatenate(
        [jnp.concatenate([nbr[c], nbr[c] - ctr[c]], axis=1)
         for c in range(3)], axis=0)
    v = jnp.dot(big.astype(jnp.bfloat16), w3t_ref[...],
                preferred_element_type=jnp.float32)
    return [v[c * _R:(c + 1) * _R, :] for c in range(3)]


def _vn_block(v, dmat_t_ref, mv, gb):
    n = jnp.sqrt((v[0] * v[0] + v[1] * v[1] + v[2] * v[2]) + 1e-12)
    nbn = (gb[0:1, :] * (n - mv[0:1, :])) / jnp.sqrt(mv[1:2, :] + 1e-5) \
        + gb[1:2, :]
    vb = [vc / n * nbn for vc in v]
    big = jnp.concatenate(vb, axis=0)
    d = jnp.dot(big.astype(jnp.bfloat16), dmat_t_ref[...],
                preferred_element_type=jnp.float32)
    dc = [d[c * _R:(c + 1) * _R, :] for c in range(3)]
    dot = (vb[0] * dc[0] + vb[1] * dc[1]) + vb[2] * dc[2]
    dsq = (dc[0] * dc[0] + dc[1] * dc[1]) + dc[2] * dc[2]
    w = dot / (dsq + 1e-8)
    pos = dot >= 0
    return [0.2 * vb[c] + 0.8 * jnp.where(pos, vb[c], vb[c] - w * dc[c])
            for c in range(3)]


def _norm3(v):
    return jnp.sqrt(((v[0] * v[0] + v[1] * v[1]) + v[2] * v[2]) + 1e-12)


def _to_u4(nbr_ref, ctr_ref, w3t_ref, d3t_ref, mv3_ref, bn3_ref, w4t_ref):
    v3 = _lin3(nbr_ref, ctr_ref, w3t_ref)
    o3 = _vn_block(v3, d3t_ref, mv3_ref[...], bn3_ref[...])
    big = jnp.concatenate(o3, axis=0)
    u = jnp.dot(big.astype(jnp.bfloat16), w4t_ref[...],
                preferred_element_type=jnp.float32)
    return [u[c * _R:(c + 1) * _R, :] for c in range(3)]


def _n3_kernel(nbr_ref, ctr_ref, w3t_ref, out_ref):
    v3 = _lin3(nbr_ref, ctr_ref, w3t_ref)
    out_ref[0] = jnp.reshape(_norm3(v3), (_S, _PT, 128))


def _n4_kernel(nbr_ref, ctr_ref, w3t_ref, d3t_ref, mv3_ref, bn3_ref,
               w4t_ref, out_ref):
    u = _to_u4(nbr_ref, ctr_ref, w3t_ref, d3t_ref, mv3_ref, bn3_ref,
               w4t_ref)
    out_ref[0] = jnp.reshape(_norm3(u), (_S, _PT, 256))


def _final_kernel(nbr_ref, ctr_ref, w3t_ref, d3t_ref, mv3_ref, bn3_ref,
                  w4t_ref, d4t_ref, mv4_ref, bn4_ref, out_ref):
    u = _to_u4(nbr_ref, ctr_ref, w3t_ref, d3t_ref, mv3_ref, bn3_ref,
               w4t_ref)
    o4 = _vn_block(u, d4t_ref, mv4_ref[...], bn4_ref[...])
    oc = [jnp.reshape(o4[c], (_S, _PT, 256)) for c in range(3)]
    n2 = (oc[0] * oc[0] + oc[1] * oc[1]) + oc[2] * oc[2]
    best = n2[0]
    bv = [oc[c][0] for c in range(3)]
    for s in range(1, _S):
        upd = n2[s] > best
        best = jnp.where(upd, n2[s], best)
        bv = [jnp.where(upd, oc[c][s], bv[c]) for c in range(3)]
    for c in range(3):
        out_ref[0, c] = bv[c]


def _bfr(x):
    return x.astype(jnp.bfloat16).astype(jnp.float32)


def _pool_basis_kernel(pool_ref, wp_ref, out_ref):
    p = [pool_ref[0, c] for c in range(3)]
    n2 = (p[0] * p[0] + p[1] * p[1]) + p[2] * p[2]
    m = jnp.max(n2, axis=0, keepdims=True)
    sub = jax.lax.broadcasted_iota(jnp.int32, (_P, 256), 0)
    idxm = jnp.where(n2 == m, sub, _BIGI)
    win = jnp.min(idxm, axis=0, keepdims=True)
    onehot = sub == win
    out_ref[...] = jnp.zeros_like(out_ref)
    for c in range(3):
        feat = _bfr(jnp.sum(jnp.where(onehot, p[c], 0.0), axis=0,
                            keepdims=True))
        v1 = jnp.sum(feat * wp_ref[0:1, :], axis=1, keepdims=True)
        v2 = jnp.sum(feat * wp_ref[1:2, :], axis=1, keepdims=True)
        out_ref[0, c:c + 1, :] = jnp.broadcast_to(v1, (1, 128))
        out_ref[0, c + 3:c + 4, :] = jnp.broadcast_to(v2, (1, 128))


def _pad8(rows):
    return jnp.concatenate(
        rows + [jnp.zeros((8 - len(rows), rows[0].shape[1]), jnp.float32)],
        axis=0)


@jax.jit
def kernel(x, W1, D1, g1, b1, W2, D2, g2, b2, W3, D3, g3, b3, W4, D4, g4,
           b4, Wp):
    del W1, D1, g1, b1, W2, D2, g2, b2
    f32 = jnp.float32
    bf = jnp.bfloat16

    xt = jnp.transpose(x, (0, 2, 1))
    pts_t = xt[:, ::4, :]
    pts = jnp.transpose(pts_t, (0, 2, 1)).astype(f32)
    ctrcol = pts[:, :, ::2, None]
    ctr5 = jnp.broadcast_to(ctrcol[:, :, None, :, :], (_B, 3, _S, _P, 1))

    bn3 = _pad8([g3[None, :], b3[None, :]])
    bn4 = _pad8([g4[None, :], b4[None, :]])
    wpb = Wp.astype(bf).astype(f32)
    wp = _pad8([wpb[0][None, :], wpb[1][None, :]])
    w3t = jnp.transpose(W3).astype(bf)
    d3t = jnp.transpose(D3).astype(bf)
    w4t = jnp.transpose(W4).astype(bf)
    d4t = jnp.transpose(D4).astype(bf)

    nbr = pl.pallas_call(
        _knn_kernel,
        grid=(_B,),
        in_specs=[
            pl.BlockSpec((1, 3, _N), lambda b: (b, 0, 0)),
            pl.BlockSpec((1, 3, _P, 1), lambda b: (b, 0, 0, 0)),
        ],
        out_specs=pl.BlockSpec((1, 3, _P * _S, 1), lambda b: (b, 0, 0, 0)),
        out_shape=jax.ShapeDtypeStruct((_B, 3, _P * _S, 1), f32),
    )(pts, ctrcol)

    nbr5 = jnp.reshape(nbr, (_B, 3, _S, _P, 1))
    nbr_spec = pl.BlockSpec((1, 3, _S, _PT, 1), lambda b, t: (b, 0, 0, t, 0))
    w3t_spec = pl.BlockSpec((2, 128), lambda b, t: (0, 0))
    d3t_spec = pl.BlockSpec((128, 128), lambda b, t: (0, 0))
    w4t_spec = pl.BlockSpec((128, 256), lambda b, t: (0, 0))
    d4t_spec = pl.BlockSpec((256, 256), lambda b, t: (0, 0))
    r128_spec = pl.BlockSpec((8, 128), lambda b, t: (0, 0))
    r256_spec = pl.BlockSpec((8, 256), lambda b, t: (0, 0))

    n3d = pl.pallas_call(
        _n3_kernel,
        grid=(_B, _NT),
        in_specs=[nbr_spec, nbr_spec, w3t_spec],
        out_specs=pl.BlockSpec((1, _S, _PT, 128), lambda b, t: (b, 0, t, 0)),
        out_shape=jax.ShapeDtypeStruct((_B, _S, _P, 128), f32),
    )(nbr5, ctr5, w3t)

    n3s = jnp.transpose(n3d, (0, 3, 2, 1))
    mv3 = _pad8([jnp.mean(n3s, axis=(0, 2, 3))[None, :],
                 jnp.var(n3s, axis=(0, 2, 3))[None, :]])

    n4d = pl.pallas_call(
        _n4_kernel,
        grid=(_B, _NT),
        in_specs=[nbr_spec, nbr_spec, w3t_spec, d3t_spec, r128_spec,
                  r128_spec, w4t_spec],
        out_specs=pl.BlockSpec((1, _S, _PT, 256), lambda b, t: (b, 0, t, 0)),
        out_shape=jax.ShapeDtypeStruct((_B, _S, _P, 256), f32),
    )(nbr5, ctr5, w3t, d3t, mv3, bn3, w4t)

    n4s = jnp.transpose(n4d, (0, 3, 2, 1))
    mv4 = _pad8([jnp.mean(n4s, axis=(0, 2, 3))[None, :],
                 jnp.var(n4s, axis=(0, 2, 3))[None, :]])

    pooled = pl.pallas_call(
        _final_kernel,
        grid=(_B, _NT),
        in_specs=[nbr_spec, nbr_spec, w3t_spec, d3t_spec, r128_spec,
                  r128_spec, w4t_spec, d4t_spec, r256_spec, r256_spec],
        out_specs=pl.BlockSpec((1, 3, _PT, 256), lambda b, t: (b, 0, t, 0)),
        out_shape=jax.ShapeDtypeStruct((_B, 3, _P, 256), f32),
    )(nbr5, ctr5, w3t, d3t, mv3, bn3, w4t, d4t, mv4, bn4)

    basis = pl.pallas_call(
        _pool_basis_kernel,
        grid=(_B,),
        in_specs=[
            pl.BlockSpec((1, 3, _P, 256), lambda b: (b, 0, 0, 0)),
            pl.BlockSpec((8, 256), lambda b: (0, 0)),
        ],
        out_specs=pl.BlockSpec((1, 8, 128), lambda b: (b, 0, 0)),
        out_shape=jax.ShapeDtypeStruct((_B, 8, 128), f32),
    )(pooled, wp)

    v1 = basis[:, 0:3, 0]
    v2 = basis[:, 3:6, 0]
    u1 = v1 / jnp.maximum(jnp.linalg.norm(v1, axis=-1, keepdims=True), 1e-8)
    u2_raw = v2 - jnp.sum(v2 * u1, axis=-1, keepdims=True) * u1
    u2 = u2_raw / jnp.maximum(jnp.linalg.norm(u2_raw, axis=-1, keepdims=True),
                              1e-6)
    u3 = jnp.cross(u1, u2)
    R = jnp.stack([u1, u2, u3], axis=-1)
    det = jnp.linalg.det(R)
    sign = jnp.sign(det)[:, None, None]
    R = jnp.concatenate([R[:, :, 0:1], R[:, :, 1:2], R[:, :, 2:3] * sign],
                        axis=-1)
    return (R, v1, v2)
